# baseline (speedup 1.0000x reference)
import functools

import jax
import jax.numpy as jnp
from jax import lax
from jax.experimental import pallas as pl
from jax.experimental.pallas import tpu as pltpu
from jax.experimental.pallas import tpu_sc as plsc

NUM_CORES = 2
NUM_SUBCORES = 16
NW = NUM_CORES * NUM_SUBCORES
BATCH = 16384
EMBED = 64
HIDDEN = 128
BPW = BATCH // NW
CHUNK = 128
NCHUNK = BPW // CHUNK

@functools.lru_cache(maxsize=1)
def _get_sc_gather():
    mesh = plsc.VectorSubcoreMesh(
        core_axis_name="c", subcore_axis_name="s",
        num_cores=NUM_CORES, num_subcores=NUM_SUBCORES)

    @functools.partial(
        pl.kernel,
        out_type=(jax.ShapeDtypeStruct((BATCH, EMBED), jnp.float32),
                  jax.ShapeDtypeStruct((BATCH, EMBED), jnp.float32)),
        mesh=mesh,
        scratch_types=(
            pltpu.VMEM((BPW,), jnp.int32),
            pltpu.VMEM((BPW,), jnp.int32),
            pltpu.VMEM((BPW, EMBED), jnp.float32),
            pltpu.VMEM((BPW, EMBED), jnp.float32),
            pltpu.SemaphoreType.DMA,
        ),
        compiler_params=pltpu.CompilerParams(use_tc_tiling_on_sc=False),
    )
    def sc_gather(uid_hbm, vid_hbm, utab_hbm, vtab_hbm, u_out, v_out,
                  uidx, vidx, urows, vrows, sem):
        wid = lax.axis_index("s") * NUM_CORES + lax.axis_index("c")
        base = wid * BPW
        pltpu.sync_copy(uid_hbm.at[pl.ds(base, BPW)], uidx)
        pltpu.sync_copy(vid_hbm.at[pl.ds(base, BPW)], vidx)
        copies = []
        for j in range(NCHUNK):
            s = pl.ds(j * CHUNK, CHUNK)
            copies.append(pltpu.async_copy(utab_hbm.at[uidx.at[s]], urows.at[s], sem))
            copies.append(pltpu.async_copy(vtab_hbm.at[vidx.at[s]], vrows.at[s], sem))
        for c in copies:
            c.wait()
        pltpu.sync_copy(urows, u_out.at[pl.ds(base, BPW)])
        pltpu.sync_copy(vrows, v_out.at[pl.ds(base, BPW)])

    return sc_gather


BM = 2048


def _mlp_body(xu_ref, xv_ref, w1_ref, b1_ref, w2_ref, b2_ref, out_ref):
    h = jnp.dot(xu_ref[...], w1_ref[0:EMBED, :],
                preferred_element_type=jnp.float32)
    h = h + jnp.dot(xv_ref[...], w1_ref[EMBED:2 * EMBED, :],
                    preferred_element_type=jnp.float32)
    h = jnp.maximum(h + b1_ref[...], 0.0)
    o = jnp.dot(h, w2_ref[...], preferred_element_type=jnp.float32) + b2_ref[0, 0]
    out_ref[...] = jax.nn.sigmoid(o)


_mlp = pl.pallas_call(
    _mlp_body,
    grid=(BATCH // BM,),
    in_specs=[
        pl.BlockSpec((BM, EMBED), lambda i: (i, 0)),
        pl.BlockSpec((BM, EMBED), lambda i: (i, 0)),
        pl.BlockSpec((2 * EMBED, HIDDEN), lambda i: (0, 0)),
        pl.BlockSpec((1, HIDDEN), lambda i: (0, 0)),
        pl.BlockSpec((HIDDEN, 1), lambda i: (0, 0)),
        pl.BlockSpec((1, 1), lambda i: (0, 0)),
    ],
    out_specs=pl.BlockSpec((BM, 1), lambda i: (i, 0)),
    out_shape=jax.ShapeDtypeStruct((BATCH, 1), jnp.float32),
)


def kernel(user_id, video_id, user_table, video_table, W1, b1, W2, b2):
    u, v = _get_sc_gather()(user_id, video_id, user_table, video_table)
    return _mlp(u, v, W1, b1.reshape(1, HIDDEN), W2, b2.reshape(1, 1))
